# Initial kernel scaffold; baseline (speedup 1.0000x reference)
import jax
import jax.numpy as jnp
from jax.experimental import pallas as pl


def kernel(nodes, neigh_idx, features, weight):
    raise NotImplementedError("write your pallas kernel here")


# speedup vs baseline: 10.0192x; 10.0192x over previous
import functools

import jax
import jax.numpy as jnp
from jax import lax
from jax.experimental import pallas as pl
from jax.experimental.pallas import tpu as pltpu
from jax.experimental.pallas import tpu_sc as plsc

B = 16384
DEG = 32
D = 128
LANES = 16
NC, NS = 2, 16
NW = NC * NS
BPW = B // NW
IPR = 128
NPC = IPR // DEG
NCHUNK = BPW // NPC
IDXROWS = BPW * DEG // IPR
NV = D // LANES


def _tanh(x):
    e = jnp.exp(-2.0 * jnp.abs(x))
    return jnp.sign(x) * (1.0 - e) / (1.0 + e)


@functools.partial(
    pl.kernel,
    out_type=jax.ShapeDtypeStruct((B, D), jnp.float32),
    mesh=plsc.VectorSubcoreMesh(core_axis_name="c", subcore_axis_name="s"),
    scratch_types=[
        pltpu.VMEM((IDXROWS, IPR), jnp.int32),
        pltpu.VMEM((IPR, D), jnp.float32),
        pltpu.VMEM((IPR, D), jnp.float32),
        pltpu.VMEM((BPW, D), jnp.float32),
        pltpu.SemaphoreType.DMA,
        pltpu.SemaphoreType.DMA,
    ],
)
def _ego_encode(idx_hbm, feat_hbm, out_hbm, idx_v, rows0, rows1, ostage,
                sem0, sem1):
    wid = lax.axis_index("s") * NC + lax.axis_index("c")
    pltpu.sync_copy(idx_hbm.at[wid], idx_v)

    rows = (rows0, rows1)
    sems = (sem0, sem1)

    def start(g, buf, sem):
        pltpu.async_copy(feat_hbm.at[idx_v.at[g]], buf, sem)

    def wait(buf, sem):
        pltpu.make_async_copy(feat_hbm.at[idx_v.at[0]], buf, sem).wait()

    def reduce_chunk(g, buf):
        for n in range(NPC):
            rbase = n * DEG

            def body(k, accs, rbase=rbase):
                r = rbase + 2 * k
                return tuple(
                    accs[j]
                    + buf[r, pl.ds(j * LANES, LANES)]
                    + buf[r + 1, pl.ds(j * LANES, LANES)]
                    for j in range(NV)
                )

            accs = lax.fori_loop(
                0, DEG // 2, body,
                tuple(jnp.zeros((LANES,), jnp.float32) for _ in range(NV)),
            )
            orow = g * NPC + n
            for j in range(NV):
                ostage[orow, pl.ds(j * LANES, LANES)] = _tanh(
                    accs[j] * (1.0 / DEG))

    start(0, rows0, sem0)

    def outer(i, carry):
        for b in range(2):
            g = 2 * i + b

            @pl.when(g + 1 < NCHUNK)
            def _(g=g, b=b):
                start(g + 1, rows[1 - b], sems[1 - b])

            wait(rows[b], sems[b])
            reduce_chunk(g, rows[b])
        return carry

    lax.fori_loop(0, NCHUNK // 2, outer, 0)
    pltpu.sync_copy(ostage, out_hbm.at[pl.ds(wid * BPW, BPW)])


def kernel(nodes, neigh_idx, features, weight):
    del nodes, weight
    idx = neigh_idx.reshape(NW, IDXROWS, IPR)
    return _ego_encode(idx, features)


# speedup vs baseline: 10.3263x; 1.0307x over previous
import functools

import jax
import jax.numpy as jnp
from jax import lax
from jax.experimental import pallas as pl
from jax.experimental.pallas import tpu as pltpu
from jax.experimental.pallas import tpu_sc as plsc

B = 16384
DEG = 32
D = 128
LANES = 16
NC, NS = 2, 16
NW = NC * NS
BPW = B // NW
IPR = 128
NPC = IPR // DEG
NCHUNK = BPW // NPC
IDXROWS = BPW * DEG // IPR
NV = D // LANES


def _tanh(x):
    e = jnp.exp(-2.0 * jnp.abs(x))
    return jnp.sign(x) * (1.0 - e) / (1.0 + e)


@functools.partial(
    pl.kernel,
    out_type=jax.ShapeDtypeStruct((B, D), jnp.float32),
    mesh=plsc.VectorSubcoreMesh(core_axis_name="c", subcore_axis_name="s"),
    scratch_types=[
        pltpu.VMEM((IDXROWS, IPR), jnp.int32),
        pltpu.VMEM((IPR, D), jnp.float32),
        pltpu.VMEM((IPR, D), jnp.float32),
        pltpu.VMEM((IPR, D), jnp.float32),
        pltpu.VMEM((IPR, D), jnp.float32),
        pltpu.VMEM((BPW // 2, D), jnp.float32),
        pltpu.SemaphoreType.DMA,
        pltpu.SemaphoreType.DMA,
        pltpu.SemaphoreType.DMA,
        pltpu.SemaphoreType.DMA,
    ],
)
def _ego_encode(idx_hbm, feat_hbm, out_hbm, idx_v, rows0, rows1, rows2, rows3,
                ostage, sem0, sem1, sem2, sem3):
    wid = lax.axis_index("s") * NC + lax.axis_index("c")
    pltpu.sync_copy(idx_hbm.at[wid], idx_v)

    rows = (rows0, rows1, rows2, rows3)
    sems = (sem0, sem1, sem2, sem3)
    NBUF = 4

    def start(g, buf, sem):
        pltpu.async_copy(feat_hbm.at[idx_v.at[g]], buf, sem)

    def wait(buf, sem):
        pltpu.make_async_copy(feat_hbm.at[idx_v.at[0]], buf, sem).wait()

    def reduce_chunk(g, buf):
        for n in range(NPC):
            rbase = n * DEG

            def body(k, accs, rbase=rbase):
                r = rbase + 2 * k
                return tuple(
                    accs[j]
                    + buf[r, pl.ds(j * LANES, LANES)]
                    + buf[r + 1, pl.ds(j * LANES, LANES)]
                    for j in range(NV)
                )

            accs = lax.fori_loop(
                0, DEG // 2, body,
                tuple(jnp.zeros((LANES,), jnp.float32) for _ in range(NV)),
            )
            half = NCHUNK // 2
            orow = jnp.where(g < half, g, g - half) * NPC + n
            for j in range(NV):
                ostage[orow, pl.ds(j * LANES, LANES)] = _tanh(
                    accs[j] * (1.0 / DEG))

    for p in range(NBUF - 1):
        start(p, rows[p], sems[p])

    def outer(i, carry):
        for b in range(NBUF):
            g = NBUF * i + b
            nxt = (b + NBUF - 1) % NBUF

            @pl.when(g + NBUF - 1 < NCHUNK)
            def _(g=g, nxt=nxt):
                start(g + NBUF - 1, rows[nxt], sems[nxt])

            wait(rows[b], sems[b])
            reduce_chunk(g, rows[b])

            @pl.when(g == NCHUNK // 2 - 1)
            def _(g=g):
                pltpu.sync_copy(ostage,
                                out_hbm.at[pl.ds(wid * BPW, BPW // 2)])
        return carry

    lax.fori_loop(0, NCHUNK // NBUF, outer, 0)
    pltpu.sync_copy(ostage, out_hbm.at[pl.ds(wid * BPW + BPW // 2, BPW // 2)])


def kernel(nodes, neigh_idx, features, weight):
    del nodes, weight
    idx = neigh_idx.reshape(NW, IDXROWS, IPR)
    return _ego_encode(idx, features)
